# Initial kernel scaffold; baseline (speedup 1.0000x reference)
import jax
import jax.numpy as jnp
from jax.experimental import pallas as pl


def kernel(tokens, emb):
    raise NotImplementedError("write your pallas kernel here")


# speedup vs baseline: 1.4926x; 1.4926x over previous
import jax
import jax.numpy as jnp
from jax.experimental import pallas as pl

_BN = 256


def _body(tok_ref, emb_ref, out_ref):
    out_ref[...] = tok_ref[...] + emb_ref[...][None, :, :]


def kernel(tokens, emb):
    B, N, C = tokens.shape
    return pl.pallas_call(
        _body,
        grid=(N // _BN, B),
        in_specs=[
            pl.BlockSpec((1, _BN, C), lambda i, b: (b, i, 0)),
            pl.BlockSpec((_BN, C), lambda i, b: (i, 0)),
        ],
        out_specs=pl.BlockSpec((1, _BN, C), lambda i, b: (b, i, 0)),
        out_shape=jax.ShapeDtypeStruct(tokens.shape, tokens.dtype),
    )(tokens, emb)
